# baseline (speedup 1.0000x reference)
import functools

import jax
import jax.numpy as jnp
from jax import lax
from jax.experimental import pallas as pl
from jax.experimental.pallas import tpu as pltpu
from jax.experimental.pallas import tpu_sc as plsc

EMBED_D = 316
D_PAD = 320
B_TOTAL = 4096 * 20

NUM_CORES = 2
NUM_SUBCORES = 16
NW = NUM_CORES * NUM_SUBCORES
B_PER_W = B_TOTAL // NW
CHUNK = 128
N_CHUNKS = B_PER_W // CHUNK

_mesh = plsc.VectorSubcoreMesh(core_axis_name="c", subcore_axis_name="s")


@functools.partial(
    pl.kernel,
    mesh=_mesh,
    out_type=jax.ShapeDtypeStruct((B_TOTAL, D_PAD), jnp.float32),
    scratch_types=[
        pltpu.VMEM((CHUNK,), jnp.int32),
        pltpu.VMEM((CHUNK,), jnp.int32),
        pltpu.VMEM((CHUNK, D_PAD), jnp.float32),
        pltpu.VMEM((CHUNK, D_PAD), jnp.float32),
        pltpu.SemaphoreType.DMA,
        pltpu.SemaphoreType.DMA,
    ],
    compiler_params=pltpu.CompilerParams(use_tc_tiling_on_sc=False),
)
def _gather_kernel(idx_hbm, table_hbm, out_hbm, idx0_v, idx1_v,
                   rows0_v, rows1_v, sem0, sem1):
    wid = lax.axis_index("s") * NUM_CORES + lax.axis_index("c")
    base = wid * B_PER_W
    idxs = (idx0_v, idx1_v)
    bufs = (rows0_v, rows1_v)
    sems = (sem0, sem1)

    pltpu.sync_copy(idx_hbm.at[pl.ds(base, CHUNK)], idx0_v)
    pltpu.async_copy(table_hbm.at[idx0_v], bufs[0], sems[0])

    for c in range(N_CHUNKS):
        cur = c % 2
        nxt = (c + 1) % 2
        if c + 1 < N_CHUNKS:
            pltpu.sync_copy(
                idx_hbm.at[pl.ds(base + (c + 1) * CHUNK, CHUNK)], idxs[nxt]
            )
            pltpu.async_copy(table_hbm.at[idxs[nxt]], bufs[nxt], sems[nxt])
        pltpu.make_async_copy(table_hbm.at[idxs[cur]], bufs[cur], sems[cur]).wait()
        pltpu.sync_copy(bufs[cur], out_hbm.at[pl.ds(base + c * CHUNK, CHUNK)])


def kernel(indices, table):
    flat_idx = indices.reshape(-1)
    table_pad = jnp.pad(table, ((0, 0), (0, D_PAD - table.shape[1])))
    out = _gather_kernel(flat_idx, table_pad)
    return out[:, : table.shape[1]].reshape(indices.shape + (table.shape[1],))
